# baseline (speedup 1.0000x reference)
import functools

import jax
import jax.numpy as jnp
from jax import lax
from jax.experimental import pallas as pl
from jax.experimental.pallas import tpu as pltpu
from jax.experimental.pallas import tpu_sc as plsc

_NUM_ITEMS = 1000000
_EMBED_DIM = 64
_BATCH = 4096
_HIST = 50
_B = _BATCH * _HIST

_INFO = plsc.get_sparse_core_info()
_NC = _INFO.num_cores
_NS = _INFO.num_subcores
_NW = _NC * _NS
_BPW = _B // _NW
_CHUNK = 128
_NCHUNK = _BPW // _CHUNK

_mesh = plsc.VectorSubcoreMesh(core_axis_name="c", subcore_axis_name="s")


@functools.partial(
    pl.kernel,
    mesh=_mesh,
    out_type=jax.ShapeDtypeStruct((_B, _EMBED_DIM), jnp.float32),
    scratch_types=[
        pltpu.VMEM((_BPW,), jnp.int32),
        pltpu.VMEM((_CHUNK, _EMBED_DIM), jnp.float32),
        pltpu.SemaphoreType.DMA,
    ],
    compiler_params=pltpu.CompilerParams(use_tc_tiling_on_sc=False),
)
def _gather_kernel(idx_hbm, table_hbm, out_hbm, idx_v, rows_v, sem):
    wid = lax.axis_index("s") * _NC + lax.axis_index("c")
    base = wid * _BPW
    pltpu.sync_copy(idx_hbm.at[pl.ds(base, _BPW)], idx_v)

    def body(j, carry):
        off = j * _CHUNK
        pltpu.async_copy(
            table_hbm.at[idx_v.at[pl.ds(off, _CHUNK)]], rows_v, sem
        ).wait()
        pltpu.sync_copy(rows_v, out_hbm.at[pl.ds(base + off, _CHUNK)])
        return carry

    lax.fori_loop(0, _NCHUNK, body, 0)


def kernel(x, table):
    flat_idx = x.reshape(_B).astype(jnp.int32)
    out = _gather_kernel(flat_idx, table)
    return out.reshape(_BATCH, _HIST, _EMBED_DIM)


# speedup vs baseline: 1.0438x; 1.0438x over previous
import functools

import jax
import jax.numpy as jnp
from jax import lax
from jax.experimental import pallas as pl
from jax.experimental.pallas import tpu as pltpu
from jax.experimental.pallas import tpu_sc as plsc

_NUM_ITEMS = 1000000
_EMBED_DIM = 64
_BATCH = 4096
_HIST = 50
_B = _BATCH * _HIST

_INFO = plsc.get_sparse_core_info()
_NC = _INFO.num_cores
_NS = _INFO.num_subcores
_NW = _NC * _NS
_BPW = _B // _NW
_CHUNK = 128
_NCHUNK = _BPW // _CHUNK
_NBUF = 10
_LEAD = 5
_NSTEP = (_NCHUNK + _LEAD + _NBUF - 1) // _NBUF

_mesh = plsc.VectorSubcoreMesh(core_axis_name="c", subcore_axis_name="s")


@functools.partial(
    pl.kernel,
    mesh=_mesh,
    out_type=jax.ShapeDtypeStruct((_B, _EMBED_DIM), jnp.float32),
    scratch_types=(
        [pltpu.VMEM((_BPW,), jnp.int32)]
        + [pltpu.VMEM((_CHUNK, _EMBED_DIM), jnp.float32) for _ in range(_NBUF)]
        + [pltpu.SemaphoreType.DMA for _ in range(2 * _NBUF)]
    ),
    compiler_params=pltpu.CompilerParams(use_tc_tiling_on_sc=False),
)
def _gather_kernel(idx_hbm, table_hbm, out_hbm, idx_v, *scratch):
    bufs = scratch[:_NBUF]
    gsem = scratch[_NBUF:2 * _NBUF]
    osem = scratch[2 * _NBUF:]
    wid = lax.axis_index("s") * _NC + lax.axis_index("c")
    base = wid * _BPW
    pltpu.sync_copy(idx_hbm.at[pl.ds(base, _BPW)], idx_v)

    def step(t, carry):
        for b in range(_NBUF):
            j = t * _NBUF + b
            jm = j - _LEAD
            bb = (b - _LEAD) % _NBUF

            @pl.when(jnp.logical_and(t >= 1, j < _NCHUNK))
            def _():
                pltpu.make_async_copy(
                    bufs[b],
                    out_hbm.at[pl.ds(base + (j - _NBUF) * _CHUNK, _CHUNK)],
                    osem[b],
                ).wait()

            @pl.when(j < _NCHUNK)
            def _():
                pltpu.async_copy(
                    table_hbm.at[idx_v.at[pl.ds(j * _CHUNK, _CHUNK)]],
                    bufs[b],
                    gsem[b],
                )

            @pl.when(jnp.logical_and(jm >= 0, jm < _NCHUNK))
            def _():
                pltpu.make_async_copy(
                    table_hbm.at[idx_v.at[pl.ds(jm * _CHUNK, _CHUNK)]],
                    bufs[bb],
                    gsem[bb],
                ).wait()
                pltpu.async_copy(
                    bufs[bb],
                    out_hbm.at[pl.ds(base + jm * _CHUNK, _CHUNK)],
                    osem[bb],
                )
        return carry

    lax.fori_loop(0, _NSTEP, step, 0)

    for b in range(_NBUF):
        k = _NCHUNK - _NBUF + b
        pltpu.make_async_copy(
            bufs[b],
            out_hbm.at[pl.ds(base + k * _CHUNK, _CHUNK)],
            osem[b],
        ).wait()


def kernel(x, table):
    flat_idx = x.reshape(_B).astype(jnp.int32)
    out = _gather_kernel(flat_idx, table)
    return out.reshape(_BATCH, _HIST, _EMBED_DIM)


# speedup vs baseline: 1.1046x; 1.0582x over previous
import functools

import jax
import jax.numpy as jnp
from jax import lax
from jax.experimental import pallas as pl
from jax.experimental.pallas import tpu as pltpu
from jax.experimental.pallas import tpu_sc as plsc

_NUM_ITEMS = 1000000
_EMBED_DIM = 64
_PAD_ROWS = 1000064
_PAD_DIM = 128
_BATCH = 4096
_HIST = 50
_B = _BATCH * _HIST

_INFO = plsc.get_sparse_core_info()
_NC = _INFO.num_cores
_NS = _INFO.num_subcores
_NW = _NC * _NS
_BPW = _B // _NW
_CHUNK = 128
_NCHUNK = _BPW // _CHUNK
_NBUF = 6
_LEAD = 3
_NSTEP = (_NCHUNK + _LEAD + _NBUF - 1) // _NBUF

_mesh = plsc.VectorSubcoreMesh(core_axis_name="c", subcore_axis_name="s")


@functools.partial(
    pl.kernel,
    mesh=_mesh,
    out_type=jax.ShapeDtypeStruct((_B, _EMBED_DIM), jnp.float32),
    scratch_types=(
        [pltpu.VMEM((_BPW,), jnp.int32)]
        + [pltpu.VMEM((_CHUNK, _PAD_DIM), jnp.float32) for _ in range(_NBUF)]
        + [pltpu.SemaphoreType.DMA for _ in range(2 * _NBUF)]
    ),
    compiler_params=pltpu.CompilerParams(use_tc_tiling_on_sc=False),
)
def _gather_kernel(idx_hbm, table_hbm, out_hbm, idx_v, *scratch):
    bufs = scratch[:_NBUF]
    gsem = scratch[_NBUF:2 * _NBUF]
    osem = scratch[2 * _NBUF:]
    wid = lax.axis_index("s") * _NC + lax.axis_index("c")
    base = wid * _BPW
    pltpu.sync_copy(idx_hbm.at[pl.ds(base, _BPW)], idx_v)

    def step(t, carry):
        for b in range(_NBUF):
            j = t * _NBUF + b
            jm = j - _LEAD
            bb = (b - _LEAD) % _NBUF

            @pl.when(jnp.logical_and(t >= 1, j < _NCHUNK))
            def _():
                pltpu.make_async_copy(
                    bufs[b].at[:, pl.ds(0, _EMBED_DIM)],
                    out_hbm.at[pl.ds(base + (j - _NBUF) * _CHUNK, _CHUNK)],
                    osem[b],
                ).wait()

            @pl.when(j < _NCHUNK)
            def _():
                pltpu.async_copy(
                    table_hbm.at[idx_v.at[pl.ds(j * _CHUNK, _CHUNK)]],
                    bufs[b],
                    gsem[b],
                )

            @pl.when(jnp.logical_and(jm >= 0, jm < _NCHUNK))
            def _():
                pltpu.make_async_copy(
                    table_hbm.at[idx_v.at[pl.ds(jm * _CHUNK, _CHUNK)]],
                    bufs[bb],
                    gsem[bb],
                ).wait()
                pltpu.async_copy(
                    bufs[bb].at[:, pl.ds(0, _EMBED_DIM)],
                    out_hbm.at[pl.ds(base + jm * _CHUNK, _CHUNK)],
                    osem[bb],
                )
        return carry

    lax.fori_loop(0, _NSTEP, step, 0)

    for b in range(_NBUF):
        k = _NCHUNK - _NBUF + b
        pltpu.make_async_copy(
            bufs[b].at[:, pl.ds(0, _EMBED_DIM)],
            out_hbm.at[pl.ds(base + k * _CHUNK, _CHUNK)],
            osem[b],
        ).wait()


def kernel(x, table):
    flat_idx = x.reshape(_B).astype(jnp.int32)
    padded = jnp.pad(
        table,
        ((0, _PAD_ROWS - _NUM_ITEMS), (0, _PAD_DIM - _EMBED_DIM)),
    )
    out = _gather_kernel(flat_idx, padded)
    return out.reshape(_BATCH, _HIST, _EMBED_DIM)
